# baseline (speedup 1.0000x reference)
import functools

import jax
import jax.numpy as jnp
from jax import lax
from jax.experimental import pallas as pl
from jax.experimental.pallas import tpu as pltpu
from jax.experimental.pallas import tpu_sc as plsc

N = 10000
NPAD = 10240
D = 128
E = 320000
NUM_CLASSES = 2
NC, NS, LANES = 2, 16, 16
NW = NC * NS
CB = 128
NCHUNK = 80
EPW = NCHUNK * CB
EPAD = NW * EPW
RPW = NPAD // NS
NEG = -3.0e38

_MESH = dict(core_axis_name="c", subcore_axis_name="s",
             num_cores=NC, num_subcores=NS)


def _wid():
    return lax.axis_index("s") * NC + lax.axis_index("c")


def _combine_tiles(local_v, stage_sh, out_hbm, acc_v, tmp_v, op):
    c = lax.axis_index("c")
    s = lax.axis_index("s")
    pltpu.sync_copy(local_v, stage_sh.at[s])
    plsc.subcore_barrier()
    base = s * RPW
    pltpu.sync_copy(stage_sh.at[0, pl.ds(base, RPW)], acc_v)
    for t in range(1, NS):
        pltpu.sync_copy(stage_sh.at[t, pl.ds(base, RPW)], tmp_v)

        def red(i, _):
            sl = pl.ds(i * LANES, LANES)
            acc_v[sl] = op(acc_v[sl], tmp_v[sl])
            return 0

        lax.fori_loop(0, RPW // LANES, red, 0)
    pltpu.sync_copy(acc_v, out_hbm.at[c, pl.ds(base, RPW)])


_SEC = 16


def _agg_body(x_hbm, src_hbm, dst_hbm, zrows_hbm, out_hbm,
              agg_s, src_v, dst_v, rows, gsems, ssems):
    s = lax.axis_index("s")
    c = lax.axis_index("c")
    wid = _wid()
    pltpu.sync_copy(zrows_hbm, agg_s.at[pl.ds(s * RPW, RPW)])
    plsc.subcore_barrier()

    def section(q, _):
        pltpu.sync_copy(src_hbm.at[wid, pl.ds(q * _SEC, _SEC)], src_v)
        pltpu.sync_copy(dst_hbm.at[wid, pl.ds(q * _SEC, _SEC)], dst_v)

        def pair(p, _):
            j0 = 2 * p
            for b in range(2):
                pltpu.async_copy(x_hbm.at[src_v.at[j0 + b]], rows[b],
                                 gsems[b])
            for b in range(2):
                pltpu.make_async_copy(x_hbm.at[src_v.at[j0 + b]], rows[b],
                                      gsems[b]).wait()
                pltpu.async_copy(rows[b], agg_s.at[dst_v.at[j0 + b]],
                                 ssems[b], add=True)
            for b in range(2):
                pltpu.make_async_copy(rows[b], agg_s.at[dst_v.at[j0 + b]],
                                      ssems[b]).wait()
            return 0

        lax.fori_loop(0, _SEC // 2, pair, 0)
        return 0

    lax.fori_loop(0, NCHUNK // _SEC, section, 0)
    plsc.subcore_barrier()
    pltpu.sync_copy(agg_s.at[pl.ds(s * RPW, RPW)],
                    out_hbm.at[c, pl.ds(s * RPW, RPW)])


_agg = functools.partial(
    pl.kernel,
    out_type=jax.ShapeDtypeStruct((NC, NPAD, D), jnp.float32),
    mesh=plsc.VectorSubcoreMesh(**_MESH),
    compiler_params=pltpu.CompilerParams(needs_layout_passes=False),
    scratch_types=[
        pltpu.VMEM_SHARED((NPAD, D), jnp.float32),
        pltpu.VMEM((_SEC, CB), jnp.int32),
        pltpu.VMEM((_SEC, CB), jnp.int32),
        [pltpu.VMEM((CB, D), jnp.float32)] * 2,
        [pltpu.SemaphoreType.DMA] * 2,
        [pltpu.SemaphoreType.DMA] * 2,
    ],
)(_agg_body)


def _ft_body(agg_ref, ww_ref, wb_ref, fw_ref, o_ref):
    a = agg_ref[0] + agg_ref[1]
    z = jnp.dot(a, ww_ref[...], preferred_element_type=jnp.float32)
    z = z + wb_ref[...]
    o_ref[...] = jnp.dot(z, fw_ref[...], preferred_element_type=jnp.float32)


def _ft(agg2, W_w, W_b2, fc_w):
    br = 1024
    return pl.pallas_call(
        _ft_body,
        grid=(NPAD // br,),
        in_specs=[
            pl.BlockSpec((NC, br, D), lambda i: (0, i, 0)),
            pl.BlockSpec((D, D), lambda i: (0, 0)),
            pl.BlockSpec((1, D), lambda i: (0, 0)),
            pl.BlockSpec((D, D), lambda i: (0, 0)),
        ],
        out_specs=pl.BlockSpec((br, D), lambda i: (i, 0)),
        out_shape=jax.ShapeDtypeStruct((NPAD, D), jnp.float32),
    )(agg2, W_w, W_b2, fc_w)


def _edge_compute(j, fts_v, ftd_v, dst_v, e_v, m_l):
    inv_sqrt_d = jnp.float32(0.08838834764831845)
    lanes_iota = lax.iota(jnp.int32, LANES)

    def group(g, _):
        ev = jnp.zeros((LANES,), jnp.float32)
        for i in range(LANES):
            r = g * LANES + i
            acc = fts_v[r, pl.ds(0, LANES)] * ftd_v[r, pl.ds(0, LANES)]
            for k in range(1, D // LANES):
                sl = pl.ds(k * LANES, LANES)
                acc = acc + fts_v[r, sl] * ftd_v[r, sl]
            ev = jnp.where(lanes_iota == i, jnp.sum(acc) * inv_sqrt_d, ev)
        e_v[j, pl.ds(g * LANES, LANES)] = ev
        dv = dst_v[j, pl.ds(g * LANES, LANES)]
        cur = plsc.load_gather(m_l, [dv])
        need = ev > cur

        def body(nd):
            plsc.store_scatter(m_l, [dv], ev, mask=nd)
            cur2 = plsc.load_gather(m_l, [dv])
            return nd & (ev > cur2)

        lax.while_loop(jnp.any, body, need)
        return 0

    lax.fori_loop(0, CB // LANES, group, 0)


def _edge_body(ft_hbm, src_hbm, dst_hbm, neg_hbm, e_hbm, m_hbm,
               m_sh, src_v, dst_v, e_v, m_l, acc_v, tmp_v,
               fts, ftd, sems, semd):
    wid = _wid()
    pltpu.sync_copy(neg_hbm, m_l)
    pltpu.sync_copy(src_hbm.at[wid], src_v)
    pltpu.sync_copy(dst_hbm.at[wid], dst_v)

    def fire(j, b):
        pltpu.async_copy(ft_hbm.at[src_v.at[j]], fts[b], sems[b])
        pltpu.async_copy(ft_hbm.at[dst_v.at[j]], ftd[b], semd[b])

    def drain(j, b):
        pltpu.make_async_copy(ft_hbm.at[src_v.at[j]], fts[b], sems[b]).wait()
        pltpu.make_async_copy(ft_hbm.at[dst_v.at[j]], ftd[b], semd[b]).wait()

    fire(0, 0)

    def pair(p, _):
        j0 = 2 * p
        fire(j0 + 1, 1)
        drain(j0, 0)
        _edge_compute(j0, fts[0], ftd[0], dst_v, e_v, m_l)

        @pl.when(j0 + 2 < NCHUNK)
        def _():
            fire(j0 + 2, 0)

        drain(j0 + 1, 1)
        _edge_compute(j0 + 1, fts[1], ftd[1], dst_v, e_v, m_l)
        return 0

    lax.fori_loop(0, NCHUNK // 2, pair, 0)
    pltpu.sync_copy(e_v, e_hbm.at[wid])
    _combine_tiles(m_l, m_sh, m_hbm, acc_v, tmp_v, jnp.maximum)


_edge = functools.partial(
    pl.kernel,
    out_type=(jax.ShapeDtypeStruct((NW, NCHUNK, CB), jnp.float32),
              jax.ShapeDtypeStruct((NC, NPAD), jnp.float32)),
    mesh=plsc.VectorSubcoreMesh(**_MESH),
    compiler_params=pltpu.CompilerParams(needs_layout_passes=False),
    scratch_types=[
        pltpu.VMEM_SHARED((NS, NPAD), jnp.float32),
        pltpu.VMEM((NCHUNK, CB), jnp.int32),
        pltpu.VMEM((NCHUNK, CB), jnp.int32),
        pltpu.VMEM((NCHUNK, CB), jnp.float32),
        pltpu.VMEM((NPAD,), jnp.float32),
        pltpu.VMEM((RPW,), jnp.float32),
        pltpu.VMEM((RPW,), jnp.float32),
        [pltpu.VMEM((CB, D), jnp.float32)] * 2,
        [pltpu.VMEM((CB, D), jnp.float32)] * 2,
        [pltpu.SemaphoreType.DMA] * 2,
        [pltpu.SemaphoreType.DMA] * 2,
    ],
)(_edge_body)


def _soft_body(e_hbm, dst_hbm, m2_hbm, z_hbm, eexp_hbm, d_hbm,
               d_sh, dst_v, e_v, x_v, m_v, d_l, acc_v, tmp_v, big_v):
    wid = _wid()
    pltpu.sync_copy(m2_hbm.at[0], m_v)
    pltpu.sync_copy(m2_hbm.at[1], big_v)

    def mx(i, _):
        sl = pl.ds(i * LANES, LANES)
        m_v[sl] = jnp.maximum(m_v[sl], big_v[sl])
        return 0

    lax.fori_loop(0, NPAD // LANES, mx, 0)
    pltpu.sync_copy(z_hbm, d_l)
    pltpu.sync_copy(e_hbm.at[wid], e_v)
    pltpu.sync_copy(dst_hbm.at[wid], dst_v)

    def chunk(j, _):
        for g in range(CB // LANES):
            sl = pl.ds(g * LANES, LANES)
            ev = e_v[j, sl]
            dv = dst_v[j, sl]
            mg = plsc.load_gather(m_v, [dv])
            xg = jnp.exp(ev - mg)
            x_v[j, sl] = xg
            plsc.addupdate_scatter(d_l, [dv], xg)
        return 0

    lax.fori_loop(0, NCHUNK, chunk, 0)
    pltpu.sync_copy(x_v, eexp_hbm.at[wid])
    _combine_tiles(d_l, d_sh, d_hbm, acc_v, tmp_v, jnp.add)


_soft = functools.partial(
    pl.kernel,
    out_type=(jax.ShapeDtypeStruct((NW, NCHUNK, CB), jnp.float32),
              jax.ShapeDtypeStruct((NC, NPAD), jnp.float32)),
    mesh=plsc.VectorSubcoreMesh(**_MESH),
    compiler_params=pltpu.CompilerParams(needs_layout_passes=False),
    scratch_types=[
        pltpu.VMEM_SHARED((NS, NPAD), jnp.float32),
        pltpu.VMEM((NCHUNK, CB), jnp.int32),
        pltpu.VMEM((NCHUNK, CB), jnp.float32),
        pltpu.VMEM((NCHUNK, CB), jnp.float32),
        pltpu.VMEM((NPAD,), jnp.float32),
        pltpu.VMEM((NPAD,), jnp.float32),
        pltpu.VMEM((RPW,), jnp.float32),
        pltpu.VMEM((RPW,), jnp.float32),
        pltpu.VMEM((NPAD,), jnp.float32),
    ],
)(_soft_body)


def _dinv_body(d_ref, o_ref):
    d = d_ref[0:1, :] + d_ref[1:2, :]
    o_ref[...] = jnp.float32(1.0) / d


def _dinv(d2):
    return pl.pallas_call(
        _dinv_body,
        in_specs=[pl.BlockSpec((NC, NPAD), lambda: (0, 0))],
        out_specs=pl.BlockSpec((1, NPAD), lambda: (0, 0)),
        out_shape=jax.ShapeDtypeStruct((1, NPAD), jnp.float32),
    )(d2)


def _coef_body(eexp_hbm, src_hbm, dst_hbm, dinv_hbm, z_hbm, c_hbm,
               c_sh, src_v, dst_v, x_v, d_v, c_l, acc_v, tmp_v):
    wid = _wid()
    pltpu.sync_copy(dinv_hbm.at[0], d_v)
    pltpu.sync_copy(z_hbm, c_l)
    pltpu.sync_copy(eexp_hbm.at[wid], x_v)
    pltpu.sync_copy(src_hbm.at[wid], src_v)
    pltpu.sync_copy(dst_hbm.at[wid], dst_v)

    def chunk(j, _):
        for g in range(CB // LANES):
            sl = pl.ds(g * LANES, LANES)
            dv = dst_v[j, sl]
            sv = src_v[j, sl]
            dg = plsc.load_gather(d_v, [dv])
            ag = x_v[j, sl] * dg
            plsc.addupdate_scatter(c_l, [sv], ag)
        return 0

    lax.fori_loop(0, NCHUNK, chunk, 0)
    _combine_tiles(c_l, c_sh, c_hbm, acc_v, tmp_v, jnp.add)


_coef = functools.partial(
    pl.kernel,
    out_type=jax.ShapeDtypeStruct((NC, NPAD), jnp.float32),
    mesh=plsc.VectorSubcoreMesh(**_MESH),
    compiler_params=pltpu.CompilerParams(needs_layout_passes=False),
    scratch_types=[
        pltpu.VMEM_SHARED((NS, NPAD), jnp.float32),
        pltpu.VMEM((NCHUNK, CB), jnp.int32),
        pltpu.VMEM((NCHUNK, CB), jnp.int32),
        pltpu.VMEM((NCHUNK, CB), jnp.float32),
        pltpu.VMEM((NPAD,), jnp.float32),
        pltpu.VMEM((NPAD,), jnp.float32),
        pltpu.VMEM((RPW,), jnp.float32),
        pltpu.VMEM((RPW,), jnp.float32),
    ],
)(_coef_body)


def _out_body(c_ref, ft_ref, lw_ref, lb_ref, o_ref):
    csum = (c_ref[0:1, :] + c_ref[1:2, :]) * jnp.float32(1.0 / N)
    lane = lax.broadcasted_iota(jnp.int32, (1, NPAD), 1)
    csum = jnp.where(lane < N, csum, jnp.float32(0.0))
    pooled = jnp.dot(csum, ft_ref[...], preferred_element_type=jnp.float32)
    o_ref[...] = (jnp.dot(pooled, lw_ref[...],
                          preferred_element_type=jnp.float32) + lb_ref[...])


def _final(c2, ft, lin_w, lin_b2):
    return pl.pallas_call(
        _out_body,
        in_specs=[
            pl.BlockSpec((NC, NPAD), lambda: (0, 0)),
            pl.BlockSpec((NPAD, D), lambda: (0, 0)),
            pl.BlockSpec((D, NUM_CLASSES), lambda: (0, 0)),
            pl.BlockSpec((1, NUM_CLASSES), lambda: (0, 0)),
        ],
        out_specs=pl.BlockSpec((1, NUM_CLASSES), lambda: (0, 0)),
        out_shape=jax.ShapeDtypeStruct((1, NUM_CLASSES), jnp.float32),
    )(c2, ft, lin_w, lin_b2)


def kernel(x, edge_index, W_w, W_b, fc_w, lin_w, lin_b):
    xp = jnp.concatenate(
        [x, jnp.zeros((NPAD - N, D), jnp.float32)], axis=0)
    pad_idx = jnp.full((EPAD - E,), NPAD - 1, jnp.int32)
    src = jnp.concatenate([edge_index[0], pad_idx]).reshape(NW, NCHUNK, CB)
    dst = jnp.concatenate([edge_index[1], pad_idx]).reshape(NW, NCHUNK, CB)
    zrows = jnp.zeros((RPW, D), jnp.float32)
    zvec = jnp.zeros((NPAD,), jnp.float32)
    negvec = jnp.full((NPAD,), NEG, jnp.float32)

    agg2 = _agg(xp, src, dst, zrows)
    ft = _ft(agg2, W_w, W_b.reshape(1, D), fc_w)
    e, m2 = _edge(ft, src, dst, negvec)
    eexp, d2 = _soft(e, dst, m2, zvec)
    dinv = _dinv(d2)
    c2 = _coef(eexp, src, dst, dinv, zvec)
    return _final(c2, ft, lin_w, lin_b.reshape(1, NUM_CLASSES))


# speedup vs baseline: 3.3026x; 3.3026x over previous
import functools

import jax
import jax.numpy as jnp
from jax import lax
from jax.experimental import pallas as pl
from jax.experimental.pallas import tpu as pltpu
from jax.experimental.pallas import tpu_sc as plsc

N = 10000
NPAD = 10240
D = 128
E = 320000
NUM_CLASSES = 2
NC, NS, LANES = 2, 16, 16
NW = NC * NS
CB = 128
NCHUNK = 80
EPW = NCHUNK * CB
EPAD = NW * EPW
RPW = NPAD // NS
NEG = -3.0e38

_MESH = dict(core_axis_name="c", subcore_axis_name="s",
             num_cores=NC, num_subcores=NS)


def _wid():
    return lax.axis_index("s") * NC + lax.axis_index("c")


def _combine_tiles(local_v, stage_sh, out_hbm, acc_v, tmp_v, op):
    c = lax.axis_index("c")
    s = lax.axis_index("s")
    pltpu.sync_copy(local_v, stage_sh.at[s])
    plsc.subcore_barrier()
    base = s * RPW
    pltpu.sync_copy(stage_sh.at[0, pl.ds(base, RPW)], acc_v)
    for t in range(1, NS):
        pltpu.sync_copy(stage_sh.at[t, pl.ds(base, RPW)], tmp_v)

        def red(i, _):
            sl = pl.ds(i * LANES, LANES)
            acc_v[sl] = op(acc_v[sl], tmp_v[sl])
            return 0

        lax.fori_loop(0, RPW // LANES, red, 0)
    pltpu.sync_copy(acc_v, out_hbm.at[c, pl.ds(base, RPW)])


_SEC = 16


def _agg_body(x_hbm, src_hbm, dst_hbm, zrows_hbm, out_hbm,
              agg_s, src_v, dst_v, rows, gsems, ssems):
    s = lax.axis_index("s")
    c = lax.axis_index("c")
    wid = _wid()
    pltpu.sync_copy(zrows_hbm, agg_s.at[pl.ds(s * RPW, RPW)])
    plsc.subcore_barrier()

    def section(q, _):
        pltpu.sync_copy(src_hbm.at[wid, pl.ds(q * _SEC, _SEC)], src_v)
        pltpu.sync_copy(dst_hbm.at[wid, pl.ds(q * _SEC, _SEC)], dst_v)

        def pair(p, _):
            j0 = 2 * p
            for b in range(2):
                pltpu.async_copy(x_hbm.at[src_v.at[j0 + b]], rows[b],
                                 gsems[b])
            for b in range(2):
                pltpu.make_async_copy(x_hbm.at[src_v.at[j0 + b]], rows[b],
                                      gsems[b]).wait()
                pltpu.async_copy(rows[b], agg_s.at[dst_v.at[j0 + b]],
                                 ssems[b], add=True)
            for b in range(2):
                pltpu.make_async_copy(rows[b], agg_s.at[dst_v.at[j0 + b]],
                                      ssems[b]).wait()
            return 0

        lax.fori_loop(0, _SEC // 2, pair, 0)
        return 0

    lax.fori_loop(0, NCHUNK // _SEC, section, 0)
    plsc.subcore_barrier()
    pltpu.sync_copy(agg_s.at[pl.ds(s * RPW, RPW)],
                    out_hbm.at[c, pl.ds(s * RPW, RPW)])


_agg = functools.partial(
    pl.kernel,
    out_type=jax.ShapeDtypeStruct((NC, NPAD, D), jnp.float32),
    mesh=plsc.VectorSubcoreMesh(**_MESH),
    compiler_params=pltpu.CompilerParams(needs_layout_passes=False),
    scratch_types=[
        pltpu.VMEM_SHARED((NPAD, D), jnp.float32),
        pltpu.VMEM((_SEC, CB), jnp.int32),
        pltpu.VMEM((_SEC, CB), jnp.int32),
        [pltpu.VMEM((CB, D), jnp.float32)] * 2,
        [pltpu.SemaphoreType.DMA] * 2,
        [pltpu.SemaphoreType.DMA] * 2,
    ],
)(_agg_body)


def _ft_body(agg_ref, ww_ref, wb_ref, fw_ref, o_ref):
    a = agg_ref[0] + agg_ref[1]
    z = jnp.dot(a, ww_ref[...], preferred_element_type=jnp.float32)
    z = z + wb_ref[...]
    o_ref[...] = jnp.dot(z, fw_ref[...], preferred_element_type=jnp.float32)


def _ft(agg2, W_w, W_b2, fc_w):
    br = 1024
    return pl.pallas_call(
        _ft_body,
        grid=(NPAD // br,),
        in_specs=[
            pl.BlockSpec((NC, br, D), lambda i: (0, i, 0)),
            pl.BlockSpec((D, D), lambda i: (0, 0)),
            pl.BlockSpec((1, D), lambda i: (0, 0)),
            pl.BlockSpec((D, D), lambda i: (0, 0)),
        ],
        out_specs=pl.BlockSpec((br, D), lambda i: (i, 0)),
        out_shape=jax.ShapeDtypeStruct((NPAD, D), jnp.float32),
    )(agg2, W_w, W_b2, fc_w)


def _edge_compute(j, fts_v, ftd_v, dst_v, e_v, m_l):
    inv_sqrt_d = jnp.float32(0.08838834764831845)
    lanes_iota = lax.iota(jnp.int32, LANES)

    def group(g, _):
        ev = jnp.zeros((LANES,), jnp.float32)
        for i in range(LANES):
            r = g * LANES + i
            acc = fts_v[r, pl.ds(0, LANES)] * ftd_v[r, pl.ds(0, LANES)]
            for k in range(1, D // LANES):
                sl = pl.ds(k * LANES, LANES)
                acc = acc + fts_v[r, sl] * ftd_v[r, sl]
            ev = jnp.where(lanes_iota == i, jnp.sum(acc) * inv_sqrt_d, ev)
        e_v[j, pl.ds(g * LANES, LANES)] = ev
        dv = dst_v[j, pl.ds(g * LANES, LANES)]
        cur = plsc.load_gather(m_l, [dv])
        need = ev > cur

        def body(nd):
            plsc.store_scatter(m_l, [dv], ev, mask=nd)
            cur2 = plsc.load_gather(m_l, [dv])
            return nd & (ev > cur2)

        lax.while_loop(jnp.any, body, need)
        return 0

    lax.fori_loop(0, CB // LANES, group, 0)


def _edge_body(ft_hbm, src_hbm, dst_hbm, neg_hbm, e_hbm, m_hbm,
               m_sh, src_v, dst_v, e_v, m_l, acc_v, tmp_v,
               fts, ftd, sems, semd):
    wid = _wid()
    pltpu.sync_copy(neg_hbm, m_l)
    pltpu.sync_copy(src_hbm.at[wid], src_v)
    pltpu.sync_copy(dst_hbm.at[wid], dst_v)

    def fire(j, b):
        pltpu.async_copy(ft_hbm.at[src_v.at[j]], fts[b], sems[b])
        pltpu.async_copy(ft_hbm.at[dst_v.at[j]], ftd[b], semd[b])

    def drain(j, b):
        pltpu.make_async_copy(ft_hbm.at[src_v.at[j]], fts[b], sems[b]).wait()
        pltpu.make_async_copy(ft_hbm.at[dst_v.at[j]], ftd[b], semd[b]).wait()

    fire(0, 0)

    def pair(p, _):
        j0 = 2 * p
        fire(j0 + 1, 1)
        drain(j0, 0)
        _edge_compute(j0, fts[0], ftd[0], dst_v, e_v, m_l)

        @pl.when(j0 + 2 < NCHUNK)
        def _():
            fire(j0 + 2, 0)

        drain(j0 + 1, 1)
        _edge_compute(j0 + 1, fts[1], ftd[1], dst_v, e_v, m_l)
        return 0

    lax.fori_loop(0, NCHUNK // 2, pair, 0)
    pltpu.sync_copy(e_v, e_hbm.at[wid])
    _combine_tiles(m_l, m_sh, m_hbm, acc_v, tmp_v, jnp.maximum)


_edge = functools.partial(
    pl.kernel,
    out_type=(jax.ShapeDtypeStruct((NW, NCHUNK, CB), jnp.float32),
              jax.ShapeDtypeStruct((NC, NPAD), jnp.float32)),
    mesh=plsc.VectorSubcoreMesh(**_MESH),
    compiler_params=pltpu.CompilerParams(needs_layout_passes=False),
    scratch_types=[
        pltpu.VMEM_SHARED((NS, NPAD), jnp.float32),
        pltpu.VMEM((NCHUNK, CB), jnp.int32),
        pltpu.VMEM((NCHUNK, CB), jnp.int32),
        pltpu.VMEM((NCHUNK, CB), jnp.float32),
        pltpu.VMEM((NPAD,), jnp.float32),
        pltpu.VMEM((RPW,), jnp.float32),
        pltpu.VMEM((RPW,), jnp.float32),
        [pltpu.VMEM((CB, D), jnp.float32)] * 2,
        [pltpu.VMEM((CB, D), jnp.float32)] * 2,
        [pltpu.SemaphoreType.DMA] * 2,
        [pltpu.SemaphoreType.DMA] * 2,
    ],
)(_edge_body)


def _soft_body(e_hbm, dst_hbm, m2_hbm, z_hbm, eexp_hbm, d_hbm,
               d_sh, dst_v, e_v, x_v, m_v, d_l, acc_v, tmp_v, big_v):
    wid = _wid()
    pltpu.sync_copy(m2_hbm.at[0], m_v)
    pltpu.sync_copy(m2_hbm.at[1], big_v)

    def mx(i, _):
        sl = pl.ds(i * LANES, LANES)
        m_v[sl] = jnp.maximum(m_v[sl], big_v[sl])
        return 0

    lax.fori_loop(0, NPAD // LANES, mx, 0)
    pltpu.sync_copy(z_hbm, d_l)
    pltpu.sync_copy(e_hbm.at[wid], e_v)
    pltpu.sync_copy(dst_hbm.at[wid], dst_v)

    def chunk(j, _):
        for g in range(CB // LANES):
            sl = pl.ds(g * LANES, LANES)
            ev = e_v[j, sl]
            dv = dst_v[j, sl]
            mg = plsc.load_gather(m_v, [dv])
            xg = jnp.exp(ev - mg)
            x_v[j, sl] = xg
            plsc.addupdate_scatter(d_l, [dv], xg)
        return 0

    lax.fori_loop(0, NCHUNK, chunk, 0)
    pltpu.sync_copy(x_v, eexp_hbm.at[wid])
    _combine_tiles(d_l, d_sh, d_hbm, acc_v, tmp_v, jnp.add)


_soft = functools.partial(
    pl.kernel,
    out_type=(jax.ShapeDtypeStruct((NW, NCHUNK, CB), jnp.float32),
              jax.ShapeDtypeStruct((NC, NPAD), jnp.float32)),
    mesh=plsc.VectorSubcoreMesh(**_MESH),
    compiler_params=pltpu.CompilerParams(needs_layout_passes=False),
    scratch_types=[
        pltpu.VMEM_SHARED((NS, NPAD), jnp.float32),
        pltpu.VMEM((NCHUNK, CB), jnp.int32),
        pltpu.VMEM((NCHUNK, CB), jnp.float32),
        pltpu.VMEM((NCHUNK, CB), jnp.float32),
        pltpu.VMEM((NPAD,), jnp.float32),
        pltpu.VMEM((NPAD,), jnp.float32),
        pltpu.VMEM((RPW,), jnp.float32),
        pltpu.VMEM((RPW,), jnp.float32),
        pltpu.VMEM((NPAD,), jnp.float32),
    ],
)(_soft_body)


def _dinv_body(d_ref, o_ref):
    d = d_ref[0:1, :] + d_ref[1:2, :]
    o_ref[...] = jnp.float32(1.0) / d


def _dinv(d2):
    return pl.pallas_call(
        _dinv_body,
        in_specs=[pl.BlockSpec((NC, NPAD), lambda: (0, 0))],
        out_specs=pl.BlockSpec((1, NPAD), lambda: (0, 0)),
        out_shape=jax.ShapeDtypeStruct((1, NPAD), jnp.float32),
    )(d2)


def _coef_body(eexp_hbm, src_hbm, dst_hbm, dinv_hbm, z_hbm, c_hbm,
               c_sh, src_v, dst_v, x_v, d_v, c_l, acc_v, tmp_v):
    wid = _wid()
    pltpu.sync_copy(dinv_hbm.at[0], d_v)
    pltpu.sync_copy(z_hbm, c_l)
    pltpu.sync_copy(eexp_hbm.at[wid], x_v)
    pltpu.sync_copy(src_hbm.at[wid], src_v)
    pltpu.sync_copy(dst_hbm.at[wid], dst_v)

    def chunk(j, _):
        for g in range(CB // LANES):
            sl = pl.ds(g * LANES, LANES)
            dv = dst_v[j, sl]
            sv = src_v[j, sl]
            dg = plsc.load_gather(d_v, [dv])
            ag = x_v[j, sl] * dg
            plsc.addupdate_scatter(c_l, [sv], ag)
        return 0

    lax.fori_loop(0, NCHUNK, chunk, 0)
    _combine_tiles(c_l, c_sh, c_hbm, acc_v, tmp_v, jnp.add)


_coef = functools.partial(
    pl.kernel,
    out_type=jax.ShapeDtypeStruct((NC, NPAD), jnp.float32),
    mesh=plsc.VectorSubcoreMesh(**_MESH),
    compiler_params=pltpu.CompilerParams(needs_layout_passes=False),
    scratch_types=[
        pltpu.VMEM_SHARED((NS, NPAD), jnp.float32),
        pltpu.VMEM((NCHUNK, CB), jnp.int32),
        pltpu.VMEM((NCHUNK, CB), jnp.int32),
        pltpu.VMEM((NCHUNK, CB), jnp.float32),
        pltpu.VMEM((NPAD,), jnp.float32),
        pltpu.VMEM((NPAD,), jnp.float32),
        pltpu.VMEM((RPW,), jnp.float32),
        pltpu.VMEM((RPW,), jnp.float32),
    ],
)(_coef_body)


def _out_body(c_ref, ft_ref, lw_ref, lb_ref, o_ref):
    csum = (c_ref[0:1, :] + c_ref[1:2, :]) * jnp.float32(1.0 / N)
    lane = lax.broadcasted_iota(jnp.int32, (1, NPAD), 1)
    csum = jnp.where(lane < N, csum, jnp.float32(0.0))
    pooled = jnp.dot(csum, ft_ref[...], preferred_element_type=jnp.float32)
    o_ref[...] = (jnp.dot(pooled, lw_ref[...],
                          preferred_element_type=jnp.float32) + lb_ref[...])


def _final(c2, ft, lin_w, lin_b2):
    return pl.pallas_call(
        _out_body,
        in_specs=[
            pl.BlockSpec((NC, NPAD), lambda: (0, 0)),
            pl.BlockSpec((NPAD, D), lambda: (0, 0)),
            pl.BlockSpec((D, NUM_CLASSES), lambda: (0, 0)),
            pl.BlockSpec((1, NUM_CLASSES), lambda: (0, 0)),
        ],
        out_specs=pl.BlockSpec((1, NUM_CLASSES), lambda: (0, 0)),
        out_shape=jax.ShapeDtypeStruct((1, NUM_CLASSES), jnp.float32),
    )(c2, ft, lin_w, lin_b2)


def kernel(x, edge_index, W_w, W_b, fc_w, lin_w, lin_b):
    xp = jnp.concatenate(
        [x, jnp.zeros((NPAD - N, D), jnp.float32)], axis=0)
    pad_idx = N + (jnp.arange(EPAD - E, dtype=jnp.int32) % (NPAD - N))
    src = jnp.concatenate([edge_index[0], pad_idx]).reshape(NW, NCHUNK, CB)
    dst = jnp.concatenate([edge_index[1], pad_idx]).reshape(NW, NCHUNK, CB)
    zrows = jnp.zeros((RPW, D), jnp.float32)
    zvec = jnp.zeros((NPAD,), jnp.float32)
    negvec = jnp.full((NPAD,), NEG, jnp.float32)

    agg2 = _agg(xp, src, dst, zrows)
    ft = _ft(agg2, W_w, W_b.reshape(1, D), fc_w)
    e, m2 = _edge(ft, src, dst, negvec)
    eexp, d2 = _soft(e, dst, m2, zvec)
    dinv = _dinv(d2)
    c2 = _coef(eexp, src, dst, dinv, zvec)
    return _final(c2, ft, lin_w, lin_b.reshape(1, NUM_CLASSES))
